# baseline (speedup 1.0000x reference)
import functools

import jax
import jax.numpy as jnp
from jax.experimental import pallas as pl
from jax.experimental.pallas import tpu as pltpu

_LN_EPS = 1e-5


def _ln_t_kernel(w_ref, gm_ref, b_ref, beta_ref, x_ref, o_ref):
    d = jnp.dot(w_ref[...], x_ref[...], preferred_element_type=jnp.float32)
    d = d + b_ref[...]
    var = jnp.dot(gm_ref[...], d * d, preferred_element_type=jnp.float32)
    r = jax.lax.rsqrt(var + _LN_EPS)
    o_ref[...] = jnp.maximum(d * r + beta_ref[...], 0.0).astype(o_ref.dtype)


@functools.partial(jax.jit, static_argnames=("col_block",))
def _encode(x, w, b, gamma, beta, *, col_block=32768):
    batch, a_dim = x.shape
    h_dim = w.shape[1]

    w_c = w - jnp.mean(w, axis=1, keepdims=True)
    b_c = b - jnp.mean(b)

    wg = (w_c * gamma[None, :]).T
    bg_col = (b_c * gamma).reshape(h_dim, 1)
    g2 = jnp.maximum(gamma * gamma, jnp.float32(1e-30))
    gm_row = (1.0 / (h_dim * g2)).reshape(1, h_dim)
    be_col = beta.reshape(h_dim, 1)

    xt = x.T

    cost = pl.CostEstimate(
        flops=2 * batch * a_dim * h_dim,
        transcendentals=batch,
        bytes_accessed=4 * (batch * (a_dim + h_dim) + a_dim * h_dim + 3 * h_dim),
    )

    cb = min(col_block, batch)
    cb = max(128, (cb // 128) * 128)
    yt = pl.pallas_call(
        _ln_t_kernel,
        out_shape=jax.ShapeDtypeStruct((h_dim, batch), jnp.float32),
        grid=(pl.cdiv(batch, cb),),
        in_specs=[
            pl.BlockSpec((h_dim, a_dim), lambda i: (0, 0)),
            pl.BlockSpec((1, h_dim), lambda i: (0, 0)),
            pl.BlockSpec((h_dim, 1), lambda i: (0, 0)),
            pl.BlockSpec((h_dim, 1), lambda i: (0, 0)),
            pl.BlockSpec((a_dim, cb), lambda i: (0, i)),
        ],
        out_specs=pl.BlockSpec((h_dim, cb), lambda i: (0, i)),
        compiler_params=pltpu.CompilerParams(
            dimension_semantics=("parallel",),
        ),
        cost_estimate=cost,
    )(wg, gm_row, bg_col, be_col, xt)
    return yt.T


def kernel(x, w, b, gamma, beta):
    return _encode(x, w, b, gamma, beta)


# speedup vs baseline: 1.0472x; 1.0472x over previous
import functools

import jax
import jax.numpy as jnp
from jax.experimental import pallas as pl
from jax.experimental.pallas import tpu as pltpu

_LN_EPS = 1e-5


def _ln_t_kernel(sub, w_ref, gm_ref, b_ref, beta_ref, x_ref, o_ref):
    cb = x_ref.shape[1]
    w_mat = w_ref[...]
    gm_row = gm_ref[...]
    b_col = b_ref[...]
    be_col = beta_ref[...]

    def chunk(i, _):
        c0 = i * sub
        xs = x_ref[:, pl.ds(c0, sub)]
        d = jnp.dot(w_mat, xs, preferred_element_type=jnp.float32)
        d = d + b_col
        var = jnp.dot(gm_row, d * d, preferred_element_type=jnp.float32)
        r = jax.lax.rsqrt(var + _LN_EPS)
        o_ref[:, pl.ds(c0, sub)] = jnp.maximum(d * r + be_col, 0.0
                                               ).astype(o_ref.dtype)
        return _

    jax.lax.fori_loop(0, cb // sub, chunk, None)


@functools.partial(jax.jit, static_argnames=("col_block",))
def _encode(x, w, b, gamma, beta, *, col_block=98304, sub_block=16384):
    batch, a_dim = x.shape
    h_dim = w.shape[1]

    w_c = w - jnp.mean(w, axis=1, keepdims=True)
    b_c = b - jnp.mean(b)

    wg = (w_c * gamma[None, :]).T
    bg_col = (b_c * gamma).reshape(h_dim, 1)
    g2 = jnp.maximum(gamma * gamma, jnp.float32(1e-30))
    gm_row = (1.0 / (h_dim * g2)).reshape(1, h_dim)
    be_col = beta.reshape(h_dim, 1)

    xt = x.T

    cost = pl.CostEstimate(
        flops=2 * batch * a_dim * h_dim,
        transcendentals=batch,
        bytes_accessed=4 * (batch * (a_dim + h_dim) + a_dim * h_dim + 3 * h_dim),
    )

    cb = min(col_block, batch)
    cb = max(128, (cb // 128) * 128)
    sub = min(sub_block, cb)
    while cb % sub:
        sub //= 2
    body = functools.partial(_ln_t_kernel, sub)
    yt = pl.pallas_call(
        body,
        out_shape=jax.ShapeDtypeStruct((h_dim, batch), jnp.float32),
        grid=(pl.cdiv(batch, cb),),
        in_specs=[
            pl.BlockSpec((h_dim, a_dim), lambda i: (0, 0)),
            pl.BlockSpec((1, h_dim), lambda i: (0, 0)),
            pl.BlockSpec((h_dim, 1), lambda i: (0, 0)),
            pl.BlockSpec((h_dim, 1), lambda i: (0, 0)),
            pl.BlockSpec((a_dim, cb), lambda i: (0, i)),
        ],
        out_specs=pl.BlockSpec((h_dim, cb), lambda i: (0, i)),
        compiler_params=pltpu.CompilerParams(
            dimension_semantics=("parallel",),
        ),
        cost_estimate=cost,
    )(wg, gm_row, bg_col, be_col, xt)
    return yt.T


def kernel(x, w, b, gamma, beta):
    return _encode(x, w, b, gamma, beta)
